# baseline (speedup 1.0000x reference)
import functools

import jax
import jax.numpy as jnp
from jax import lax
from jax.experimental import pallas as pl
from jax.experimental.pallas import tpu as pltpu
from jax.experimental.pallas import tpu_sc as plsc

EMBED_DIM = 64
NUM_WORKERS = 32
NBUF = 4
CHUNK = 320


def _sc_gather(idx_flat, table):
    n = idx_flat.shape[0]
    per_worker = n // NUM_WORKERS
    n_chunks = per_worker // CHUNK
    assert n_chunks % NBUF == 0 and n_chunks // NBUF >= 2
    mesh = plsc.VectorSubcoreMesh(core_axis_name="c", subcore_axis_name="s")

    @functools.partial(
        pl.kernel,
        mesh=mesh,
        out_type=jax.ShapeDtypeStruct((n, EMBED_DIM), jnp.float32),
        scratch_types=[
            pltpu.VMEM((per_worker,), jnp.int32),
            *[pltpu.VMEM((CHUNK, EMBED_DIM), jnp.float32) for _ in range(NBUF)],
            *[pltpu.SemaphoreType.DMA for _ in range(2 * NBUF)],
        ],
        compiler_params=pltpu.CompilerParams(use_tc_tiling_on_sc=False),
    )
    def grab(idx_hbm, table_hbm, out_hbm, idx_v, *bufs_and_sems):
        rows = bufs_and_sems[:NBUF]
        gsem = bufs_and_sems[NBUF:2 * NBUF]
        osem = bufs_and_sems[2 * NBUF:]
        wid = lax.axis_index("s") * 2 + lax.axis_index("c")
        base = wid * per_worker
        pltpu.sync_copy(idx_hbm.at[pl.ds(base, per_worker)], idx_v)

        def gather(j, b):
            pltpu.async_copy(
                table_hbm.at[idx_v.at[pl.ds(j * CHUNK, CHUNK)]], rows[b], gsem[b])

        def put(j, b):
            pltpu.async_copy(
                rows[b], out_hbm.at[pl.ds(base + j * CHUNK, CHUNK)], osem[b])

        def wait_gather(b):
            pltpu.make_async_copy(
                table_hbm.at[pl.ds(0, CHUNK)], rows[b], gsem[b]).wait()

        def wait_put(b):
            pltpu.make_async_copy(
                rows[b], out_hbm.at[pl.ds(base, CHUNK)], osem[b]).wait()

        for b in range(NBUF):
            gather(b, b)

        def body(i, carry):
            j0 = NBUF * i
            for b in range(NBUF):
                wait_gather(b)
                put(j0 + b, b)

                @pl.when(i + 1 < n_chunks // NBUF)
                def _():
                    wait_put(b)
                    gather(j0 + b + NBUF, b)

            return carry

        lax.fori_loop(0, n_chunks // NBUF, body, 0)
        for b in range(NBUF):
            wait_put(b)

    return grab(idx_flat, table)


def kernel(indices, table):
    b, h = indices.shape
    idx_flat = indices.reshape(-1).astype(jnp.int32)
    out = _sc_gather(idx_flat, table)
    return out.reshape(b, h, EMBED_DIM)
